# baseline (speedup 1.0000x reference)
import jax
import jax.numpy as jnp
from jax.experimental import pallas as pl

_N = 10000
_F_IN = 128
_HID = 32
_NC = 2
_BLK = 2000


def _cell_body(x_ref, h_ref, c_ref, W_ref, Th_ref, bias_ref,
               wci_ref, wcf_ref, wco_ref, Wlin_ref, blin_ref,
               logits_ref, H_ref, C_ref):
    xb = x_ref[...]
    hb = h_ref[...]
    cb = c_ref[...]
    g = jnp.dot(xb, W_ref[...], preferred_element_type=jnp.float32)
    g = g + jnp.dot(hb, Th_ref[...], preferred_element_type=jnp.float32)
    g = g + bias_ref[...]
    ig = jax.nn.sigmoid(g[:, 0:_HID] + wci_ref[...] * cb)
    fg = jax.nn.sigmoid(g[:, _HID:2 * _HID] + wcf_ref[...] * cb)
    tg = jnp.tanh(g[:, 2 * _HID:3 * _HID])
    Cn = fg * cb + ig * tg
    og = jax.nn.sigmoid(g[:, 3 * _HID:4 * _HID] + wco_ref[...] * Cn)
    Hn = og * jnp.tanh(Cn)
    C_ref[...] = Cn
    H_ref[...] = Hn
    logits_ref[...] = (
        jnp.dot(jnp.maximum(Hn, 0.0), Wlin_ref[...],
                preferred_element_type=jnp.float32)
        + blin_ref[...])


def kernel(x, edge_index, edge_weight, h, c,
           W_i, Th_i, cb_i, w_c_i, b_i,
           W_f, Th_f, cb_f, w_c_f, b_f,
           W_c, Th_c, cb_c, b_c,
           W_o, Th_o, cb_o, w_c_o, b_o,
           W_lin, b_lin):
    del edge_index, edge_weight
    W_all = jnp.concatenate([W_i, W_f, W_c, W_o], axis=1)
    Th_all = jnp.concatenate([Th_i, Th_f, Th_c, Th_o], axis=1)
    bias_all = jnp.concatenate([cb_i[None, :] + b_i,
                                cb_f[None, :] + b_f,
                                cb_c[None, :] + b_c,
                                cb_o[None, :] + b_o], axis=1)
    blin = b_lin[None, :]

    grid = (_N // _BLK,)
    row_spec = lambda w: pl.BlockSpec((_BLK, w), lambda i: (i, 0))
    full_spec = lambda s: pl.BlockSpec(s, lambda i: (0, 0))

    logits, H, C = pl.pallas_call(
        _cell_body,
        grid=grid,
        in_specs=[
            row_spec(_F_IN),
            row_spec(_HID),
            row_spec(_HID),
            full_spec((_F_IN, 4 * _HID)),
            full_spec((_HID, 4 * _HID)),
            full_spec((1, 4 * _HID)),
            full_spec((1, _HID)),
            full_spec((1, _HID)),
            full_spec((1, _HID)),
            full_spec((_HID, _NC)),
            full_spec((1, _NC)),
        ],
        out_specs=[
            row_spec(_NC),
            row_spec(_HID),
            row_spec(_HID),
        ],
        out_shape=[
            jax.ShapeDtypeStruct((_N, _NC), jnp.float32),
            jax.ShapeDtypeStruct((_N, _HID), jnp.float32),
            jax.ShapeDtypeStruct((_N, _HID), jnp.float32),
        ],
    )(x, h, c, W_all, Th_all, bias_all, w_c_i, w_c_f, w_c_o, W_lin, blin)
    return (logits, H, C)
